# baseline (speedup 1.0000x reference)
import functools

import jax
import jax.numpy as jnp
from jax import lax
from jax.experimental import pallas as pl
from jax.experimental.pallas import tpu as pltpu
from jax.experimental.pallas import tpu_sc as plsc

NUM_RULES = 16384
NUM_MEMBERSHIPS = 100000
MEMBERSHIP_DIM = 32

NC = 2
NS = 16
NW = NC * NS
B_PER_W = NUM_RULES // NW
BLK = 128
NBLK = B_PER_W // BLK
L = 16

NFULL = NUM_MEMBERSHIPS // BLK
PACKED_ROWS = NUM_MEMBERSHIPS // 4
TAIL_ROW = NFULL * 32
T_PER_W = (NFULL + NW - 1) // NW


def _format_body(tab3_hbm, tail_hbm, out_hbm, src_v, dst_v, tail_v):
    wid = lax.axis_index("s") * NC + lax.axis_index("c")
    lanes = lax.iota(jnp.int32, L)
    j_vecs = [lanes + L * jg for jg in range(8)]
    q_vecs = [lax.shift_right_logical(j, 2) for j in j_vecs]
    c_vecs = [lax.shift_left(jnp.bitwise_and(j, 3), 5) for j in j_vecs]

    def do_block(t, _):
        b = wid + t * NW

        @pl.when(b < NFULL)
        def _():
            pltpu.sync_copy(tab3_hbm.at[:, :, pl.ds(b * BLK, BLK)], src_v)

            @plsc.parallel_loop(0, 2 * L, unroll=2)
            def _diag(hd):
                fd = jnp.bitwise_and(lanes + hd, L - 1) + jnp.broadcast_to(
                    jnp.bitwise_and(hd, L), (L,)
                )
                kv = lax.shift_right_logical(fd, 3)
                sv = jnp.bitwise_and(fd, 7)
                vals = [
                    plsc.load_gather(src_v, [kv, sv, j_vecs[jg]])
                    for jg in range(8)
                ]
                for jg in range(8):
                    plsc.store_scatter(
                        dst_v, [q_vecs[jg], c_vecs[jg] + fd], vals[jg]
                    )

            pltpu.sync_copy(dst_v, out_hbm.at[pl.ds(b * 32, 32), :])
        return _

    lax.fori_loop(0, T_PER_W, do_block, None)

    @pl.when(wid == NW - 1)
    def _():
        pltpu.sync_copy(tail_hbm, tail_v)
        pltpu.sync_copy(tail_v, out_hbm.at[pl.ds(TAIL_ROW, 8), :])


def _gather_body(map_hbm, tab_hbm, out_hbm, m_v, idx_v, fet_a, fet_b, blk_v, sem0, sem1):
    wid = lax.axis_index("s") * NC + lax.axis_index("c")
    base = wid * B_PER_W

    pltpu.sync_copy(map_hbm.at[pl.ds(base, B_PER_W)], m_v)
    for r in range(B_PER_W // L):
        mm = m_v[pl.ds(r * L, L)]
        idx_v[pl.ds(r * L, L)] = lax.shift_right_logical(mm, 2)

    lanes = lax.iota(jnp.int32, L)

    def fetch(db):
        return pltpu.async_copy(
            tab_hbm.at[idx_v.at[pl.ds(db * BLK, BLK)]],
            fet_a if db % 2 == 0 else fet_b,
            sem0 if db % 2 == 0 else sem1,
        )

    fh = [None] * NBLK
    fh[0] = fetch(0)
    for db in range(NBLK):
        if db + 1 < NBLK:
            fh[db + 1] = fetch(db + 1)
        fh[db].wait()
        fetched_v = fet_a if db % 2 == 0 else fet_b

        offs = []
        rows = []
        for jg in range(8):
            mm = m_v[pl.ds(db * BLK + jg * L, L)]
            offs.append(lax.shift_left(jnp.bitwise_and(mm, 3), 5))
            rows.append(lanes + jg * L)
        @plsc.parallel_loop(0, 2 * L, unroll=2)
        def _diag(hd):
            fd = jnp.bitwise_and(lanes + hd, L - 1) + jnp.broadcast_to(
                jnp.bitwise_and(hd, L), (L,)
            )
            kv = lax.shift_right_logical(fd, 3)
            sv = jnp.bitwise_and(fd, 7)
            vals = [
                plsc.load_gather(fetched_v, [rows[jg], offs[jg] + fd])
                for jg in range(8)
            ]
            for jg in range(8):
                plsc.store_scatter(
                    blk_v, [kv, sv, rows[jg] + db * BLK], vals[jg]
                )

    pltpu.sync_copy(blk_v, out_hbm.at[:, :, pl.ds(base, B_PER_W)])


@jax.jit
def _run(mapping, tab3, tail):
    mesh = plsc.VectorSubcoreMesh(core_axis_name="c", subcore_axis_name="s")
    params = pltpu.CompilerParams(
        use_tc_tiling_on_sc=True, needs_layout_passes=False
    )
    table2 = pl.kernel(
        _format_body,
        out_type=jax.ShapeDtypeStruct((PACKED_ROWS, BLK), jnp.float32),
        mesh=mesh,
        scratch_types=[
            pltpu.VMEM((4, 8, BLK), jnp.float32),
            pltpu.VMEM((32, BLK), jnp.float32),
            pltpu.VMEM((8, BLK), jnp.float32),
        ],
        compiler_params=params,
    )(tab3, tail)
    return pl.kernel(
        _gather_body,
        out_type=jax.ShapeDtypeStruct((4, 8, NUM_RULES), jnp.float32),
        mesh=mesh,
        scratch_types=[
            pltpu.VMEM((B_PER_W,), jnp.int32),
            pltpu.VMEM((B_PER_W,), jnp.int32),
            pltpu.VMEM((BLK, BLK), jnp.float32),
            pltpu.VMEM((BLK, BLK), jnp.float32),
            pltpu.VMEM((4, 8, B_PER_W), jnp.float32),
            pltpu.SemaphoreType.DMA,
            pltpu.SemaphoreType.DMA,
        ],
        compiler_params=params,
    )(mapping, table2)


def kernel(x, mapping, table):
    del x
    tab3 = table.T.reshape(4, 8, NUM_MEMBERSHIPS)
    tail = table[4 * TAIL_ROW :].reshape(8, BLK)
    out3 = _run(mapping.astype(jnp.int32), tab3, tail)
    return out3.reshape(MEMBERSHIP_DIM, NUM_RULES).T.reshape(
        NUM_RULES, 1, MEMBERSHIP_DIM
    )


# speedup vs baseline: 1.0350x; 1.0350x over previous
import functools

import jax
import jax.numpy as jnp
from jax import lax
from jax.experimental import pallas as pl
from jax.experimental.pallas import tpu as pltpu
from jax.experimental.pallas import tpu_sc as plsc

NUM_RULES = 16384
NUM_MEMBERSHIPS = 100000
MEMBERSHIP_DIM = 32

NC = 2
NS = 16
NW = NC * NS
B_PER_W = NUM_RULES // NW
BLK = 128
NBLK = B_PER_W // BLK
L = 16

NFULL = NUM_MEMBERSHIPS // BLK
PACKED_ROWS = NUM_MEMBERSHIPS // 4
TAIL_ROW = NFULL * 32
WIDE = 256
NWIDE = NUM_MEMBERSHIPS // WIDE
T_PER_W = (NWIDE + NW - 1) // NW


def _format_body(tab3_hbm, tail_hbm, out_hbm, src_v, dst_v, srcl_v, dstl_v, tail_v):
    wid = lax.axis_index("s") * NC + lax.axis_index("c")
    lanes = lax.iota(jnp.int32, L)
    j_vecs = [lanes + L * jg for jg in range(WIDE // L)]
    q_vecs = [lax.shift_right_logical(j, 2) for j in j_vecs]
    c_vecs = [lax.shift_left(jnp.bitwise_and(j, 3), 5) for j in j_vecs]

    def transpose_piece(src, dst, ngroups):
        @plsc.parallel_loop(0, 2 * L, unroll=2)
        def _diag(hd):
            fd = jnp.bitwise_and(lanes + hd, L - 1) + jnp.broadcast_to(
                jnp.bitwise_and(hd, L), (L,)
            )
            kv = lax.shift_right_logical(fd, 3)
            sv = jnp.bitwise_and(fd, 7)
            vals = [
                plsc.load_gather(src, [kv, sv, j_vecs[jg]])
                for jg in range(ngroups)
            ]
            for jg in range(ngroups):
                plsc.store_scatter(dst, [q_vecs[jg], c_vecs[jg] + fd], vals[jg])

    def do_piece(t, _):
        b = wid + t * NW

        @pl.when(b < NWIDE)
        def _():
            pltpu.sync_copy(tab3_hbm.at[:, :, pl.ds(b * WIDE, WIDE)], src_v)
            transpose_piece(src_v, dst_v, WIDE // L)
            pltpu.sync_copy(dst_v, out_hbm.at[pl.ds(b * (WIDE // 4), WIDE // 4), :])
        return _

    lax.fori_loop(0, T_PER_W, do_piece, None)

    @pl.when(wid == NW - 2)
    def _():
        pltpu.sync_copy(tab3_hbm.at[:, :, pl.ds(NWIDE * WIDE, BLK)], srcl_v)
        transpose_piece(srcl_v, dstl_v, BLK // L)
        pltpu.sync_copy(dstl_v, out_hbm.at[pl.ds(NWIDE * (WIDE // 4), 32), :])

    @pl.when(wid == NW - 1)
    def _():
        pltpu.sync_copy(tail_hbm, tail_v)
        pltpu.sync_copy(tail_v, out_hbm.at[pl.ds(TAIL_ROW, 8), :])


def _gather_body(map_hbm, tab_hbm, out_hbm, m_v, idx_v, fet_a, fet_b, blk_v, sem0, sem1):
    wid = lax.axis_index("s") * NC + lax.axis_index("c")
    base = wid * B_PER_W

    pltpu.sync_copy(map_hbm.at[pl.ds(base, B_PER_W)], m_v)
    for r in range(B_PER_W // L):
        mm = m_v[pl.ds(r * L, L)]
        idx_v[pl.ds(r * L, L)] = lax.shift_right_logical(mm, 2)

    lanes = lax.iota(jnp.int32, L)

    def fetch(db):
        return pltpu.async_copy(
            tab_hbm.at[idx_v.at[pl.ds(db * BLK, BLK)]],
            fet_a if db % 2 == 0 else fet_b,
            sem0 if db % 2 == 0 else sem1,
        )

    fh = [None] * NBLK
    fh[0] = fetch(0)
    for db in range(NBLK):
        if db + 1 < NBLK:
            fh[db + 1] = fetch(db + 1)
        fh[db].wait()
        fetched_v = fet_a if db % 2 == 0 else fet_b

        offs = []
        rows = []
        for jg in range(8):
            mm = m_v[pl.ds(db * BLK + jg * L, L)]
            offs.append(lax.shift_left(jnp.bitwise_and(mm, 3), 5))
            rows.append(lanes + jg * L)
        @plsc.parallel_loop(0, 2 * L, unroll=2)
        def _diag(hd):
            fd = jnp.bitwise_and(lanes + hd, L - 1) + jnp.broadcast_to(
                jnp.bitwise_and(hd, L), (L,)
            )
            kv = lax.shift_right_logical(fd, 3)
            sv = jnp.bitwise_and(fd, 7)
            vals = [
                plsc.load_gather(fetched_v, [rows[jg], offs[jg] + fd])
                for jg in range(8)
            ]
            for jg in range(8):
                plsc.store_scatter(
                    blk_v, [kv, sv, rows[jg] + db * BLK], vals[jg]
                )

    pltpu.sync_copy(blk_v, out_hbm.at[:, :, pl.ds(base, B_PER_W)])


@jax.jit
def _run(mapping, tab3, tail):
    mesh = plsc.VectorSubcoreMesh(core_axis_name="c", subcore_axis_name="s")
    params = pltpu.CompilerParams(
        use_tc_tiling_on_sc=True, needs_layout_passes=False
    )
    table2 = pl.kernel(
        _format_body,
        out_type=jax.ShapeDtypeStruct((PACKED_ROWS, BLK), jnp.float32),
        mesh=mesh,
        scratch_types=[
            pltpu.VMEM((4, 8, WIDE), jnp.float32),
            pltpu.VMEM((WIDE // 4, BLK), jnp.float32),
            pltpu.VMEM((4, 8, BLK), jnp.float32),
            pltpu.VMEM((32, BLK), jnp.float32),
            pltpu.VMEM((8, BLK), jnp.float32),
        ],
        compiler_params=params,
    )(tab3, tail)
    return pl.kernel(
        _gather_body,
        out_type=jax.ShapeDtypeStruct((4, 8, NUM_RULES), jnp.float32),
        mesh=mesh,
        scratch_types=[
            pltpu.VMEM((B_PER_W,), jnp.int32),
            pltpu.VMEM((B_PER_W,), jnp.int32),
            pltpu.VMEM((BLK, BLK), jnp.float32),
            pltpu.VMEM((BLK, BLK), jnp.float32),
            pltpu.VMEM((4, 8, B_PER_W), jnp.float32),
            pltpu.SemaphoreType.DMA,
            pltpu.SemaphoreType.DMA,
        ],
        compiler_params=params,
    )(mapping, table2)


def kernel(x, mapping, table):
    del x
    tab3 = table.T.reshape(4, 8, NUM_MEMBERSHIPS)
    tail = table[4 * TAIL_ROW :].reshape(8, BLK)
    out3 = _run(mapping.astype(jnp.int32), tab3, tail)
    return out3.reshape(MEMBERSHIP_DIM, NUM_RULES).T.reshape(
        NUM_RULES, 1, MEMBERSHIP_DIM
    )


# speedup vs baseline: 1.1769x; 1.1371x over previous
import functools

import jax
import jax.numpy as jnp
from jax import lax
from jax.experimental import pallas as pl
from jax.experimental.pallas import tpu as pltpu
from jax.experimental.pallas import tpu_sc as plsc

NUM_RULES = 16384
NUM_MEMBERSHIPS = 100000
MEMBERSHIP_DIM = 32

NC = 2
NS = 16
NW = NC * NS
B_PER_W = NUM_RULES // NW
BLK = 128
NBLK = B_PER_W // BLK
L = 16

NFULL = NUM_MEMBERSHIPS // BLK
PACKED_ROWS = NUM_MEMBERSHIPS // 4
TAIL_ROW = NFULL * 32
WIDE = 512
NWIDE = NUM_MEMBERSHIPS // WIDE
T_PER_W = (NWIDE + NW - 1) // NW


def _format_body(tab3_hbm, tail_hbm, out_hbm, src_v, dst_v, srcl_v, dstl_v, tail_v):
    wid = lax.axis_index("s") * NC + lax.axis_index("c")
    lanes = lax.iota(jnp.int32, L)
    j_vecs = [lanes + L * jg for jg in range(WIDE // L)]
    q_vecs = [lax.shift_right_logical(j, 2) for j in j_vecs]
    c_vecs = [lax.shift_left(jnp.bitwise_and(j, 3), 5) for j in j_vecs]

    def transpose_piece(src, dst, ngroups):
        @plsc.parallel_loop(0, 2 * L, unroll=2)
        def _diag(hd):
            fd = jnp.bitwise_and(lanes + hd, L - 1) + jnp.broadcast_to(
                jnp.bitwise_and(hd, L), (L,)
            )
            kv = lax.shift_right_logical(fd, 3)
            sv = jnp.bitwise_and(fd, 7)
            for g0 in range(0, ngroups, 8):
                vals = [
                    plsc.load_gather(src, [kv, sv, j_vecs[g0 + i]])
                    for i in range(min(8, ngroups - g0))
                ]
                for i in range(min(8, ngroups - g0)):
                    jg = g0 + i
                    plsc.store_scatter(
                        dst, [q_vecs[jg], c_vecs[jg] + fd], vals[i]
                    )

    def do_piece(t, _):
        b = wid + t * NW

        @pl.when(b < NWIDE)
        def _():
            pltpu.sync_copy(tab3_hbm.at[:, :, pl.ds(b * WIDE, WIDE)], src_v)
            transpose_piece(src_v, dst_v, WIDE // L)
            pltpu.sync_copy(dst_v, out_hbm.at[pl.ds(b * (WIDE // 4), WIDE // 4), :])
        return _

    lax.fori_loop(0, T_PER_W, do_piece, None)

    @pl.when(wid == NW - 2)
    def _():
        pltpu.sync_copy(tab3_hbm.at[:, :, pl.ds(NWIDE * WIDE, BLK)], srcl_v)
        transpose_piece(srcl_v, dstl_v, BLK // L)
        pltpu.sync_copy(dstl_v, out_hbm.at[pl.ds(NWIDE * (WIDE // 4), 32), :])

    @pl.when(wid == NW - 1)
    def _():
        pltpu.sync_copy(tail_hbm, tail_v)
        pltpu.sync_copy(tail_v, out_hbm.at[pl.ds(TAIL_ROW, 8), :])


def _gather_body(map_hbm, tab_hbm, out_hbm, m_v, idx_v, fet_a, fet_b, blk_v, sem0, sem1):
    wid = lax.axis_index("s") * NC + lax.axis_index("c")
    base = wid * B_PER_W

    pltpu.sync_copy(map_hbm.at[pl.ds(base, B_PER_W)], m_v)
    for r in range(B_PER_W // L):
        mm = m_v[pl.ds(r * L, L)]
        idx_v[pl.ds(r * L, L)] = lax.shift_right_logical(mm, 2)

    lanes = lax.iota(jnp.int32, L)

    def fetch(db):
        return pltpu.async_copy(
            tab_hbm.at[idx_v.at[pl.ds(db * BLK, BLK)]],
            fet_a if db % 2 == 0 else fet_b,
            sem0 if db % 2 == 0 else sem1,
        )

    fh = [None] * NBLK
    fh[0] = fetch(0)
    for db in range(NBLK):
        if db + 1 < NBLK:
            fh[db + 1] = fetch(db + 1)
        fh[db].wait()
        fetched_v = fet_a if db % 2 == 0 else fet_b

        offs = []
        rows = []
        for jg in range(8):
            mm = m_v[pl.ds(db * BLK + jg * L, L)]
            offs.append(lax.shift_left(jnp.bitwise_and(mm, 3), 5))
            rows.append(lanes + jg * L)
        @plsc.parallel_loop(0, 2 * L, unroll=2)
        def _diag(hd):
            fd = jnp.bitwise_and(lanes + hd, L - 1) + jnp.broadcast_to(
                jnp.bitwise_and(hd, L), (L,)
            )
            kv = lax.shift_right_logical(fd, 3)
            sv = jnp.bitwise_and(fd, 7)
            vals = [
                plsc.load_gather(fetched_v, [rows[jg], offs[jg] + fd])
                for jg in range(8)
            ]
            for jg in range(8):
                plsc.store_scatter(
                    blk_v, [kv, sv, rows[jg] + db * BLK], vals[jg]
                )

    pltpu.sync_copy(blk_v, out_hbm.at[:, :, pl.ds(base, B_PER_W)])


@jax.jit
def _run(mapping, tab3, tail):
    mesh = plsc.VectorSubcoreMesh(core_axis_name="c", subcore_axis_name="s")
    params = pltpu.CompilerParams(
        use_tc_tiling_on_sc=True, needs_layout_passes=False
    )
    table2 = pl.kernel(
        _format_body,
        out_type=jax.ShapeDtypeStruct((PACKED_ROWS, BLK), jnp.float32),
        mesh=mesh,
        scratch_types=[
            pltpu.VMEM((4, 8, WIDE), jnp.float32),
            pltpu.VMEM((WIDE // 4, BLK), jnp.float32),
            pltpu.VMEM((4, 8, BLK), jnp.float32),
            pltpu.VMEM((32, BLK), jnp.float32),
            pltpu.VMEM((8, BLK), jnp.float32),
        ],
        compiler_params=params,
    )(tab3, tail)
    return pl.kernel(
        _gather_body,
        out_type=jax.ShapeDtypeStruct((4, 8, NUM_RULES), jnp.float32),
        mesh=mesh,
        scratch_types=[
            pltpu.VMEM((B_PER_W,), jnp.int32),
            pltpu.VMEM((B_PER_W,), jnp.int32),
            pltpu.VMEM((BLK, BLK), jnp.float32),
            pltpu.VMEM((BLK, BLK), jnp.float32),
            pltpu.VMEM((4, 8, B_PER_W), jnp.float32),
            pltpu.SemaphoreType.DMA,
            pltpu.SemaphoreType.DMA,
        ],
        compiler_params=params,
    )(mapping, table2)


def kernel(x, mapping, table):
    del x
    tab3 = table.T.reshape(4, 8, NUM_MEMBERSHIPS)
    tail = table[4 * TAIL_ROW :].reshape(8, BLK)
    out3 = _run(mapping.astype(jnp.int32), tab3, tail)
    return out3.reshape(MEMBERSHIP_DIM, NUM_RULES).T.reshape(
        NUM_RULES, 1, MEMBERSHIP_DIM
    )


# speedup vs baseline: 1.2603x; 1.0709x over previous
import functools

import jax
import jax.numpy as jnp
from jax import lax
from jax.experimental import pallas as pl
from jax.experimental.pallas import tpu as pltpu
from jax.experimental.pallas import tpu_sc as plsc

NUM_RULES = 16384
NUM_MEMBERSHIPS = 100000
MEMBERSHIP_DIM = 32

NC = 2
NS = 16
NW = NC * NS
B_PER_W = NUM_RULES // NW
BLK = 128
NBLK = B_PER_W // BLK
L = 16

NFULL = NUM_MEMBERSHIPS // BLK
PACKED_ROWS = NUM_MEMBERSHIPS // 4
TAIL_ROW = NFULL * 32
WIDE = 640
NWIDE = NUM_MEMBERSHIPS // WIDE
T_PER_W = (NWIDE + NW - 1) // NW


def _format_body(tab3_hbm, tail_hbm, out_hbm, src_v, dst_v, srcl_v, dstl_v, tail_v):
    wid = lax.axis_index("s") * NC + lax.axis_index("c")
    lanes = lax.iota(jnp.int32, L)
    j_vecs = [lanes + L * jg for jg in range(WIDE // L)]
    q_vecs = [lax.shift_right_logical(j, 2) for j in j_vecs]
    c_vecs = [lax.shift_left(jnp.bitwise_and(j, 3), 5) for j in j_vecs]

    def transpose_piece(src, dst, ngroups):
        @plsc.parallel_loop(0, 2 * L, unroll=2)
        def _diag(hd):
            fd = jnp.bitwise_and(lanes + hd, L - 1) + jnp.broadcast_to(
                jnp.bitwise_and(hd, L), (L,)
            )
            kv = lax.shift_right_logical(fd, 3)
            sv = jnp.bitwise_and(fd, 7)
            for g0 in range(0, ngroups, 8):
                vals = [
                    plsc.load_gather(src, [kv, sv, j_vecs[g0 + i]])
                    for i in range(min(8, ngroups - g0))
                ]
                for i in range(min(8, ngroups - g0)):
                    jg = g0 + i
                    plsc.store_scatter(
                        dst, [q_vecs[jg], c_vecs[jg] + fd], vals[i]
                    )

    def do_piece(t, _):
        b = wid + t * NW

        @pl.when(b < NWIDE)
        def _():
            pltpu.sync_copy(tab3_hbm.at[:, :, pl.ds(b * WIDE, WIDE)], src_v)
            transpose_piece(src_v, dst_v, WIDE // L)
            pltpu.sync_copy(dst_v, out_hbm.at[pl.ds(b * (WIDE // 4), WIDE // 4), :])
        return _

    lax.fori_loop(0, T_PER_W, do_piece, None)

    @pl.when(wid == NW - 2)
    def _():
        pltpu.sync_copy(tab3_hbm.at[:, :, pl.ds(NWIDE * WIDE, BLK)], srcl_v)
        transpose_piece(srcl_v, dstl_v, BLK // L)
        pltpu.sync_copy(dstl_v, out_hbm.at[pl.ds(NWIDE * (WIDE // 4), 32), :])

    @pl.when(wid == NW - 1)
    def _():
        pltpu.sync_copy(tail_hbm, tail_v)
        pltpu.sync_copy(tail_v, out_hbm.at[pl.ds(TAIL_ROW, 8), :])


def _gather_body(map_hbm, tab_hbm, out_hbm, m_v, idx_v, f0, f1, f2, f3, blk_v, s0, s1, s2, s3):
    wid = lax.axis_index("s") * NC + lax.axis_index("c")
    base = wid * B_PER_W

    pltpu.sync_copy(map_hbm.at[pl.ds(base, B_PER_W)], m_v)
    for r in range(B_PER_W // L):
        mm = m_v[pl.ds(r * L, L)]
        idx_v[pl.ds(r * L, L)] = lax.shift_right_logical(mm, 2)

    lanes = lax.iota(jnp.int32, L)

    bufs = [f0, f1, f2, f3]
    sems = [s0, s1, s2, s3]
    fh = [
        pltpu.async_copy(
            tab_hbm.at[idx_v.at[pl.ds(db * BLK, BLK)]], bufs[db], sems[db]
        )
        for db in range(NBLK)
    ]
    for db in range(NBLK):
        fh[db].wait()
        fetched_v = bufs[db]

        offs = []
        rows = []
        for jg in range(8):
            mm = m_v[pl.ds(db * BLK + jg * L, L)]
            offs.append(lax.shift_left(jnp.bitwise_and(mm, 3), 5))
            rows.append(lanes + jg * L)
        @plsc.parallel_loop(0, 2 * L, unroll=2)
        def _diag(hd):
            fd = jnp.bitwise_and(lanes + hd, L - 1) + jnp.broadcast_to(
                jnp.bitwise_and(hd, L), (L,)
            )
            kv = lax.shift_right_logical(fd, 3)
            sv = jnp.bitwise_and(fd, 7)
            vals = [
                plsc.load_gather(fetched_v, [rows[jg], offs[jg] + fd])
                for jg in range(8)
            ]
            for jg in range(8):
                plsc.store_scatter(
                    blk_v, [kv, sv, rows[jg] + db * BLK], vals[jg]
                )

    pltpu.sync_copy(blk_v, out_hbm.at[:, :, pl.ds(base, B_PER_W)])


@jax.jit
def _run(mapping, tab3, tail):
    mesh = plsc.VectorSubcoreMesh(core_axis_name="c", subcore_axis_name="s")
    params = pltpu.CompilerParams(
        use_tc_tiling_on_sc=True, needs_layout_passes=False
    )
    table2 = pl.kernel(
        _format_body,
        out_type=jax.ShapeDtypeStruct((PACKED_ROWS, BLK), jnp.float32),
        mesh=mesh,
        scratch_types=[
            pltpu.VMEM((4, 8, WIDE), jnp.float32),
            pltpu.VMEM((WIDE // 4, BLK), jnp.float32),
            pltpu.VMEM((4, 8, BLK), jnp.float32),
            pltpu.VMEM((32, BLK), jnp.float32),
            pltpu.VMEM((8, BLK), jnp.float32),
        ],
        compiler_params=params,
    )(tab3, tail)
    return pl.kernel(
        _gather_body,
        out_type=jax.ShapeDtypeStruct((4, 8, NUM_RULES), jnp.float32),
        mesh=mesh,
        scratch_types=[
            pltpu.VMEM((B_PER_W,), jnp.int32),
            pltpu.VMEM((B_PER_W,), jnp.int32),
            pltpu.VMEM((BLK, BLK), jnp.float32),
            pltpu.VMEM((BLK, BLK), jnp.float32),
            pltpu.VMEM((BLK, BLK), jnp.float32),
            pltpu.VMEM((BLK, BLK), jnp.float32),
            pltpu.VMEM((4, 8, B_PER_W), jnp.float32),
            pltpu.SemaphoreType.DMA,
            pltpu.SemaphoreType.DMA,
            pltpu.SemaphoreType.DMA,
            pltpu.SemaphoreType.DMA,
        ],
        compiler_params=params,
    )(mapping, table2)


def kernel(x, mapping, table):
    del x
    tab3 = table.T.reshape(4, 8, NUM_MEMBERSHIPS)
    tail = table[4 * TAIL_ROW :].reshape(8, BLK)
    out3 = _run(mapping.astype(jnp.int32), tab3, tail)
    return out3.reshape(MEMBERSHIP_DIM, NUM_RULES).T.reshape(
        NUM_RULES, 1, MEMBERSHIP_DIM
    )
